# baseline (speedup 1.0000x reference)
import functools

import jax
import jax.numpy as jnp
from jax import lax
from jax.experimental import pallas as pl
from jax.experimental.pallas import tpu as pltpu
from jax.experimental.pallas import tpu_sc as plsc

BS = 4096
SEQ = 16
D = 512
ROW = 4 * D

_info = plsc.get_sparse_core_info()
_NC, _NS = _info.num_cores, _info.num_subcores
_NW = _NC * _NS
_B_PER_W = BS // _NW
_CHUNK = 16
_QCHUNK = 4 * _CHUNK
_N_CHUNKS = _B_PER_W // _CHUNK

_SUBLANE = (7, 0, 1, 2)


def _sc_gather_delta(table4, idx):
    mesh = plsc.VectorSubcoreMesh(core_axis_name="c", subcore_axis_name="s")

    @functools.partial(
        pl.kernel,
        mesh=mesh,
        out_type=jax.ShapeDtypeStruct((BS * 8, D), jnp.float32),
        scratch_types=[
            pltpu.VMEM((_N_CHUNKS, _CHUNK), jnp.int32),
            pltpu.VMEM((_QCHUNK, D), jnp.float32),
            pltpu.VMEM((_QCHUNK, D), jnp.float32),
            pltpu.SemaphoreType.DMA,
            pltpu.SemaphoreType.DMA,
        ],
    )
    def gather_kernel(table_hbm, idx_hbm, out_hbm, widx_v, rows0, rows1, sem0, sem1):
        wid = lax.axis_index("s") * _NC + lax.axis_index("c")
        base = wid * _B_PER_W
        lane = lax.iota(jnp.int32, 16)
        for c in range(_N_CHUNKS):
            pltpu.sync_copy(
                idx_hbm.at[pl.ds(base + c * _CHUNK, _CHUNK)], widx_v.at[c]
            )
        bufs = (rows0, rows1)
        sems = (sem0, sem1)
        copies = [[None] * 4, [None] * 4]

        def start_gathers(c, s):
            iv = widx_v[c, :]
            for k in range(4):
                copies[s][k] = pltpu.make_async_copy(
                    table_hbm.at[iv * 4 + k],
                    bufs[s].at[pl.ds(k * _CHUNK, _CHUNK)],
                    sems[s],
                )
                copies[s][k].start()

        def drain_scatter(c, p):
            for k in range(4):
                copies[p][k].wait()
            dbase = (base + c * _CHUNK + lane) * 8
            for k in range(4):
                pltpu.sync_copy(
                    bufs[p].at[pl.ds(k * _CHUNK, _CHUNK)],
                    out_hbm.at[dbase + _SUBLANE[k]],
                )

        for c in range(_N_CHUNKS):
            start_gathers(c, c % 2)
            if c >= 1:
                drain_scatter(c - 1, (c - 1) % 2)
        drain_scatter(_N_CHUNKS - 1, (_N_CHUNKS - 1) % 2)

    return gather_kernel(table4, idx)


_B_BLK = 256


def _tc_fused_body(x_ref, d_ref, e_ref, o_ref):
    scale = (e_ref[...] * (4.0 / 22.0)).reshape(_B_BLK, 1, 1)
    d = d_ref[...] * scale
    s = lax.broadcasted_iota(jnp.int32, (_B_BLK, 8, D), 1)
    o_ref[:, 0:8, :] = x_ref[:, 0:8, :] + jnp.where(s == 7, d, 0.0)
    o_ref[:, 8:16, :] = x_ref[:, 8:16, :] + jnp.where(s < 3, d, 0.0)


def _tc_fused(x, delta8, eps2):
    return pl.pallas_call(
        _tc_fused_body,
        grid=(BS // _B_BLK,),
        in_specs=[
            pl.BlockSpec((_B_BLK, SEQ, D), lambda i: (i, 0, 0)),
            pl.BlockSpec((_B_BLK, 8, D), lambda i: (i, 0, 0)),
            pl.BlockSpec((_B_BLK, 1), lambda i: (i, 0)),
        ],
        out_specs=pl.BlockSpec((_B_BLK, SEQ, D), lambda i: (i, 0, 0)),
        out_shape=jax.ShapeDtypeStruct((BS, SEQ, D), jnp.float32),
    )(x, delta8, eps2)


def kernel(x, w, eps, log_mat_half):
    table4 = log_mat_half.reshape(log_mat_half.shape[0] * 4, D)
    delta = _sc_gather_delta(table4, w.astype(jnp.int32))
    return _tc_fused(x, delta.reshape(BS, 8, D), eps.reshape(BS, 1))


# speedup vs baseline: 6.5441x; 6.5441x over previous
import functools

import jax
import jax.numpy as jnp
from jax import lax
from jax.experimental import pallas as pl
from jax.experimental.pallas import tpu as pltpu
from jax.experimental.pallas import tpu_sc as plsc

BS = 4096
SEQ = 16
D = 512
ROW = 4 * D

_info = plsc.get_sparse_core_info()
_NC, _NS = _info.num_cores, _info.num_subcores
_NW = _NC * _NS
_B_PER_W = BS // _NW
_CHUNK = 16
_QCHUNK = 4 * _CHUNK
_N_CHUNKS = _B_PER_W // _CHUNK

_SUBLANE = (7, 0, 1, 2)


def _sc_gather_delta(table, idx):
    mesh = plsc.VectorSubcoreMesh(core_axis_name="c", subcore_axis_name="s")

    @functools.partial(
        pl.kernel,
        mesh=mesh,
        out_type=jax.ShapeDtypeStruct((BS * 8, D), jnp.float32),
        scratch_types=[
            pltpu.VMEM((_N_CHUNKS, _CHUNK), jnp.int32),
            pltpu.VMEM((_CHUNK, ROW), jnp.float32),
            pltpu.VMEM((_CHUNK, ROW), jnp.float32),
            pltpu.SemaphoreType.DMA,
            pltpu.SemaphoreType.DMA,
        ],
    )
    def gather_kernel(table_hbm, idx_hbm, out_hbm, widx_v, rows0, rows1, sem0, sem1):
        wid = lax.axis_index("s") * _NC + lax.axis_index("c")
        base = wid * _B_PER_W
        lane = lax.iota(jnp.int32, 16)
        for c in range(_N_CHUNKS):
            pltpu.sync_copy(
                idx_hbm.at[pl.ds(base + c * _CHUNK, _CHUNK)], widx_v.at[c]
            )
        bufs = (rows0, rows1)
        sems = (sem0, sem1)
        copies = [None, None]

        def start_gather(c, s):
            copies[s] = pltpu.make_async_copy(
                table_hbm.at[widx_v[c, :]], bufs[s], sems[s]
            )
            copies[s].start()

        def drain_scatter(c, p):
            copies[p].wait()
            dbase = (base + c * _CHUNK + lane) * 8
            for k in range(4):
                pltpu.sync_copy(
                    bufs[p].at[:, pl.ds(k * D, D)],
                    out_hbm.at[dbase + _SUBLANE[k]],
                )

        for c in range(_N_CHUNKS):
            start_gather(c, c % 2)
            if c >= 1:
                drain_scatter(c - 1, (c - 1) % 2)
        drain_scatter(_N_CHUNKS - 1, (_N_CHUNKS - 1) % 2)

    return gather_kernel(table, idx)


_B_BLK = 256


def _tc_fused_body(x_ref, d_ref, e_ref, o_ref):
    scale = (e_ref[...] * (4.0 / 22.0)).reshape(_B_BLK, 1, 1)
    d = d_ref[...].reshape(_B_BLK, 8, D) * scale
    s = lax.broadcasted_iota(jnp.int32, (_B_BLK, 8, D), 1)
    o_ref[:, 0:8, :] = x_ref[:, 0:8, :] + jnp.where(s == 7, d, 0.0)
    o_ref[:, 8:16, :] = x_ref[:, 8:16, :] + jnp.where(s < 3, d, 0.0)


def _tc_fused(x, delta, eps2):
    return pl.pallas_call(
        _tc_fused_body,
        grid=(BS // _B_BLK,),
        in_specs=[
            pl.BlockSpec((_B_BLK, SEQ, D), lambda i: (i, 0, 0)),
            pl.BlockSpec((8 * _B_BLK, D), lambda i: (i, 0)),
            pl.BlockSpec((_B_BLK, 1), lambda i: (i, 0)),
        ],
        out_specs=pl.BlockSpec((_B_BLK, SEQ, D), lambda i: (i, 0, 0)),
        out_shape=jax.ShapeDtypeStruct((BS, SEQ, D), jnp.float32),
    )(x, delta, eps2)


def kernel(x, w, eps, log_mat_half):
    delta = _sc_gather_delta(log_mat_half, w.astype(jnp.int32))
    return _tc_fused(x, delta, eps.reshape(BS, 1))


# speedup vs baseline: 7.0680x; 1.0801x over previous
import functools

import jax
import jax.numpy as jnp
from jax import lax
from jax.experimental import pallas as pl
from jax.experimental.pallas import tpu as pltpu
from jax.experimental.pallas import tpu_sc as plsc

BS = 4096
SEQ = 16
D = 512
ROW = 4 * D
HALF = BS // 2

_info = plsc.get_sparse_core_info()
_NC, _NS = _info.num_cores, _info.num_subcores
_NW = _NC * _NS
_B_PER_W = HALF // _NW
_CHUNK = 16
_N_CHUNKS = _B_PER_W // _CHUNK


def _sc_gather(table, idx):
    mesh = plsc.VectorSubcoreMesh(core_axis_name="c", subcore_axis_name="s")

    @functools.partial(
        pl.kernel,
        mesh=mesh,
        out_type=jax.ShapeDtypeStruct((HALF, ROW), jnp.float32),
        scratch_types=[
            pltpu.VMEM((_N_CHUNKS, _CHUNK), jnp.int32),
            pltpu.VMEM((_CHUNK, ROW), jnp.float32),
            pltpu.VMEM((_CHUNK, ROW), jnp.float32),
            pltpu.SemaphoreType.DMA,
            pltpu.SemaphoreType.DMA,
        ],
    )
    def gather_kernel(table_hbm, idx_hbm, out_hbm, idx_v, rows0, rows1, sem0, sem1):
        wid = lax.axis_index("s") * _NC + lax.axis_index("c")
        base = wid * _B_PER_W
        for c in range(_N_CHUNKS):
            pltpu.sync_copy(idx_hbm.at[pl.ds(base + c * _CHUNK, _CHUNK)], idx_v.at[c])
        bufs = (rows0, rows1)
        sems = (sem0, sem1)
        copies = [None, None]
        for c in range(_N_CHUNKS):
            s = c % 2
            copies[s] = pltpu.make_async_copy(
                table_hbm.at[idx_v[c, :]], bufs[s], sems[s]
            )
            copies[s].start()
            if c >= 1:
                p = (c - 1) % 2
                copies[p].wait()
                pltpu.sync_copy(
                    bufs[p], out_hbm.at[pl.ds(base + (c - 1) * _CHUNK, _CHUNK)]
                )
        last = (_N_CHUNKS - 1) % 2
        copies[last].wait()
        pltpu.sync_copy(
            bufs[last], out_hbm.at[pl.ds(base + (_N_CHUNKS - 1) * _CHUNK, _CHUNK)]
        )

    return gather_kernel(table, idx)


_B_BLK = 256
_HBLKS = HALF // _B_BLK


def _tc_body(x_ref, w_ref, e_ref, o_ref):
    o_ref[...] = x_ref[...]
    wk = w_ref[...].reshape(_B_BLK, 4, D)
    scale = (e_ref[...] * (4.0 / 22.0)).reshape(_B_BLK, 1, 1)
    o_ref[:, 7:11, :] = x_ref[:, 7:11, :] + wk * scale


def _tc_body_aliased(o1_ref, x_ref, w_ref, e_ref, o_ref):
    del o1_ref
    _tc_body(x_ref, w_ref, e_ref, o_ref)


def _tc_add_half0(x, walks_h, eps2):
    return pl.pallas_call(
        _tc_body,
        grid=(_HBLKS,),
        in_specs=[
            pl.BlockSpec((_B_BLK, SEQ, D), lambda i: (i, 0, 0)),
            pl.BlockSpec((_B_BLK, ROW), lambda i: (i, 0)),
            pl.BlockSpec((_B_BLK, 1), lambda i: (i, 0)),
        ],
        out_specs=pl.BlockSpec((_B_BLK, SEQ, D), lambda i: (i, 0, 0)),
        out_shape=jax.ShapeDtypeStruct((BS, SEQ, D), jnp.float32),
    )(x, walks_h, eps2)


def _tc_add_half1(prev, x, walks_h, eps2):
    off = _HBLKS
    return pl.pallas_call(
        _tc_body_aliased,
        grid=(_HBLKS,),
        in_specs=[
            pl.BlockSpec(memory_space=pl.ANY),
            pl.BlockSpec((_B_BLK, SEQ, D), lambda i: (i + off, 0, 0)),
            pl.BlockSpec((_B_BLK, ROW), lambda i: (i, 0)),
            pl.BlockSpec((_B_BLK, 1), lambda i: (i + off, 0)),
        ],
        out_specs=pl.BlockSpec((_B_BLK, SEQ, D), lambda i: (i + off, 0, 0)),
        out_shape=jax.ShapeDtypeStruct((BS, SEQ, D), jnp.float32),
        input_output_aliases={0: 0},
    )(prev, x, walks_h, eps2)


def kernel(x, w, eps, log_mat_half):
    w = w.astype(jnp.int32)
    eps2 = eps.reshape(BS, 1)
    walks0 = _sc_gather(log_mat_half, lax.slice(w, (0,), (HALF,)))
    walks1 = _sc_gather(log_mat_half, lax.slice(w, (HALF,), (BS,)))
    out = _tc_add_half0(x, walks0, eps2)
    out = _tc_add_half1(out, x, walks1, eps2)
    return out


# speedup vs baseline: 7.1148x; 1.0066x over previous
import functools

import jax
import jax.numpy as jnp
from jax import lax
from jax.experimental import pallas as pl
from jax.experimental.pallas import tpu as pltpu
from jax.experimental.pallas import tpu_sc as plsc

BS = 4096
SEQ = 16
D = 512
ROW = 4 * D
SPLIT = 1024

_info = plsc.get_sparse_core_info()
_NC, _NS = _info.num_cores, _info.num_subcores
_NW = _NC * _NS
_CHUNK = 16


def _sc_gather(table, idx, n):
    mesh = plsc.VectorSubcoreMesh(core_axis_name="c", subcore_axis_name="s")
    b_per_w = n // _NW
    n_chunks = b_per_w // _CHUNK

    @functools.partial(
        pl.kernel,
        mesh=mesh,
        out_type=jax.ShapeDtypeStruct((n, ROW), jnp.float32),
        scratch_types=[
            pltpu.VMEM((n_chunks, _CHUNK), jnp.int32),
            pltpu.VMEM((_CHUNK, ROW), jnp.float32),
            pltpu.VMEM((_CHUNK, ROW), jnp.float32),
            pltpu.SemaphoreType.DMA,
            pltpu.SemaphoreType.DMA,
        ],
    )
    def gather_kernel(table_hbm, idx_hbm, out_hbm, idx_v, rows0, rows1, sem0, sem1):
        wid = lax.axis_index("s") * _NC + lax.axis_index("c")
        base = wid * b_per_w
        for c in range(n_chunks):
            pltpu.sync_copy(idx_hbm.at[pl.ds(base + c * _CHUNK, _CHUNK)], idx_v.at[c])
        bufs = (rows0, rows1)
        sems = (sem0, sem1)
        copies = [None, None]
        for c in range(n_chunks):
            s = c % 2
            copies[s] = pltpu.make_async_copy(
                table_hbm.at[idx_v[c, :]], bufs[s], sems[s]
            )
            copies[s].start()
            if c >= 1:
                p = (c - 1) % 2
                copies[p].wait()
                pltpu.sync_copy(
                    bufs[p], out_hbm.at[pl.ds(base + (c - 1) * _CHUNK, _CHUNK)]
                )
        last = (n_chunks - 1) % 2
        copies[last].wait()
        pltpu.sync_copy(
            bufs[last], out_hbm.at[pl.ds(base + (n_chunks - 1) * _CHUNK, _CHUNK)]
        )

    return gather_kernel(table, idx)


_B_BLK = 256


def _tc_body(x_ref, w_ref, e_ref, o_ref):
    o_ref[...] = x_ref[...]
    wk = w_ref[...].reshape(_B_BLK, 4, D)
    scale = (e_ref[...] * (4.0 / 22.0)).reshape(_B_BLK, 1, 1)
    o_ref[:, 7:11, :] = x_ref[:, 7:11, :] + wk * scale


def _tc_body_aliased(o1_ref, x_ref, w_ref, e_ref, o_ref):
    del o1_ref
    _tc_body(x_ref, w_ref, e_ref, o_ref)


def _tc_add_part0(x, walks_h, eps2, n):
    return pl.pallas_call(
        _tc_body,
        grid=(n // _B_BLK,),
        in_specs=[
            pl.BlockSpec((_B_BLK, SEQ, D), lambda i: (i, 0, 0)),
            pl.BlockSpec((_B_BLK, ROW), lambda i: (i, 0)),
            pl.BlockSpec((_B_BLK, 1), lambda i: (i, 0)),
        ],
        out_specs=pl.BlockSpec((_B_BLK, SEQ, D), lambda i: (i, 0, 0)),
        out_shape=jax.ShapeDtypeStruct((BS, SEQ, D), jnp.float32),
    )(x, walks_h, eps2)


def _tc_add_part1(prev, x, walks_h, eps2, start, n):
    off = start // _B_BLK
    return pl.pallas_call(
        _tc_body_aliased,
        grid=(n // _B_BLK,),
        in_specs=[
            pl.BlockSpec(memory_space=pl.ANY),
            pl.BlockSpec((_B_BLK, SEQ, D), lambda i: (i + off, 0, 0)),
            pl.BlockSpec((_B_BLK, ROW), lambda i: (i, 0)),
            pl.BlockSpec((_B_BLK, 1), lambda i: (i + off, 0)),
        ],
        out_specs=pl.BlockSpec((_B_BLK, SEQ, D), lambda i: (i + off, 0, 0)),
        out_shape=jax.ShapeDtypeStruct((BS, SEQ, D), jnp.float32),
        input_output_aliases={0: 0},
    )(prev, x, walks_h, eps2)


def kernel(x, w, eps, log_mat_half):
    w = w.astype(jnp.int32)
    eps2 = eps.reshape(BS, 1)
    walks0 = _sc_gather(log_mat_half, lax.slice(w, (0,), (SPLIT,)), SPLIT)
    walks1 = _sc_gather(log_mat_half, lax.slice(w, (SPLIT,), (BS,)), BS - SPLIT)
    out = _tc_add_part0(x, walks0, eps2, SPLIT)
    out = _tc_add_part1(out, x, walks1, eps2, SPLIT, BS - SPLIT)
    return out
